# baseline (speedup 1.0000x reference)
import functools
import math

import jax
import jax.numpy as jnp
from jax import lax
from jax.experimental import pallas as pl
from jax.experimental.pallas import tpu as pltpu
from jax.experimental.pallas import tpu_sc as plsc

N_NODES = 10000
N_EDGES = 160000
HIDDEN = 128
HEADS = 2
NUM_LAYERS = 2
RATIO = 0.5
NEG_SLOPE = 0.2
NEG_BIG = -1e30

_NC, _NS = 2, 16
_NW = _NC * _NS
_NPAD = 10240
_RBLK = 1000
_EBLK = 2000

_sc_mesh = functools.partial(
    plsc.VectorSubcoreMesh, core_axis_name="c", subcore_axis_name="s")


def _split_chunks(total):
    per = total // _NW
    b = min(per, 128)
    return per, b, per // b, per % b


@functools.cache
def _gather_rows_kernel(E, D):
    per = E // _NW
    B = min(per, 128)
    ngroups = per // (2 * B)
    rem = per - ngroups * 2 * B
    rfull = rem // B
    tail = rem % B

    scratch = [
        pltpu.VMEM((per,), jnp.int32),
        pltpu.VMEM((2 * B, D), jnp.float32),
        pltpu.SemaphoreType.DMA,
    ]

    @functools.partial(
        pl.kernel,
        out_type=jax.ShapeDtypeStruct((E, D), jnp.float32),
        mesh=_sc_mesh(),
        scratch_types=scratch,
    )
    def k(table_hbm, idx_hbm, out_hbm, idx_all, rows_v, sem):
        wid = lax.axis_index("s") * _NC + lax.axis_index("c")
        base = wid * per
        pltpu.sync_copy(idx_hbm.at[pl.ds(base, per)], idx_all)

        def grp(g, carry):
            off = g * 2 * B
            h1 = pltpu.async_copy(table_hbm.at[idx_all.at[pl.ds(off, B)]],
                                  rows_v.at[pl.ds(0, B)], sem)
            h2 = pltpu.async_copy(table_hbm.at[idx_all.at[pl.ds(off + B, B)]],
                                  rows_v.at[pl.ds(B, B)], sem)
            h1.wait()
            h2.wait()
            pltpu.sync_copy(rows_v, out_hbm.at[pl.ds(base + off, 2 * B)])
            return carry

        lax.fori_loop(0, ngroups, grp, 0)
        off = ngroups * 2 * B
        if rfull:
            pltpu.async_copy(table_hbm.at[idx_all.at[pl.ds(off, B)]],
                             rows_v.at[pl.ds(0, B)], sem).wait()
            pltpu.sync_copy(rows_v.at[pl.ds(0, B)],
                            out_hbm.at[pl.ds(base + off, B)])
            off += B
        if tail:
            pltpu.async_copy(table_hbm.at[idx_all.at[pl.ds(off, tail)]],
                             rows_v.at[pl.ds(0, tail)], sem).wait()
            pltpu.sync_copy(rows_v.at[pl.ds(0, tail)],
                            out_hbm.at[pl.ds(base + off, tail)])

    return k


def _gather_rows(table, idx):
    E = idx.shape[0]
    return _gather_rows_kernel(E, table.shape[1])(table, idx)


@functools.cache
def _scatter_add_kernel(E, D):
    per, B, nfull, tail = _split_chunks(E)
    rpt = _NPAD // _NS
    zb = 128
    nz = rpt // zb

    scratch = [
        pltpu.VMEM((B,), jnp.int32),
        pltpu.VMEM((B, D), jnp.float32),
        pltpu.VMEM((max(tail, 8),), jnp.int32),
        pltpu.VMEM((max(tail, 8), D), jnp.float32),
        pltpu.VMEM((zb, D), jnp.float32),
        pltpu.VMEM_SHARED((_NPAD, D), jnp.float32),
        pltpu.SemaphoreType.DMA,
    ]
    assert tail in (0, max(tail, 8))

    @functools.partial(
        pl.kernel,
        out_type=jax.ShapeDtypeStruct((_NC, _NPAD, D), jnp.float32),
        mesh=_sc_mesh(),
        scratch_types=scratch,
    )
    def k(vals_hbm, idx_hbm, out_hbm, idx_v, rows_v, idx_t, rows_t,
          zero_v, acc_sh, sem):
        cid = lax.axis_index("c")
        sid = lax.axis_index("s")
        wid = sid * _NC + cid
        base = wid * per

        zv = jnp.zeros((16,), jnp.float32)

        def zrow(r, carry):
            def zcol(cc, carry2):
                zero_v[r, pl.ds(cc * 16, 16)] = zv
                return carry2
            return lax.fori_loop(0, D // 16, zcol, carry)

        lax.fori_loop(0, zb, zrow, 0)
        for z in range(nz):
            pltpu.sync_copy(zero_v,
                            acc_sh.at[pl.ds(sid * rpt + z * zb, zb)])
        plsc.subcore_barrier()

        def step(c, carry):
            off = base + c * B
            pltpu.sync_copy(idx_hbm.at[pl.ds(off, B)], idx_v)
            pltpu.sync_copy(vals_hbm.at[pl.ds(off, B)], rows_v)
            pltpu.sync_copy(rows_v, acc_sh.at[idx_v], add=True)
            return carry

        lax.fori_loop(0, nfull, step, 0)
        if tail:
            off = base + nfull * B
            pltpu.sync_copy(idx_hbm.at[pl.ds(off, tail)], idx_t)
            pltpu.sync_copy(vals_hbm.at[pl.ds(off, tail)], rows_t)
            pltpu.sync_copy(rows_t, acc_sh.at[idx_t], add=True)
        plsc.subcore_barrier()

        for z in range(nz):
            r0 = sid * rpt + z * zb
            pltpu.sync_copy(acc_sh.at[pl.ds(r0, zb)],
                            out_hbm.at[cid, pl.ds(r0, zb)])

    return k


def _scatter_add(vals, idx):
    parts = _scatter_add_kernel(idx.shape[0], vals.shape[1])(vals, idx)
    return parts


@functools.cache
def _gather16_kernel(E):
    per, B, nfull, tail = _split_chunks(E)
    rpt = _NPAD // _NS

    scratch = [
        pltpu.VMEM((B,), jnp.int32),
        pltpu.VMEM((B, 16), jnp.float32),
        pltpu.VMEM((max(tail, 8),), jnp.int32),
        pltpu.VMEM((max(tail, 8), 16), jnp.float32),
        pltpu.VMEM_SHARED((_NPAD, 16), jnp.float32),
        pltpu.SemaphoreType.DMA,
    ]

    @functools.partial(
        pl.kernel,
        out_type=jax.ShapeDtypeStruct((E, 16), jnp.float32),
        mesh=_sc_mesh(),
        scratch_types=scratch,
    )
    def k(tab_hbm, idx_hbm, out_hbm, idx_v, rows_v, idx_t, rows_t,
          tab_sh, sem):
        sid = lax.axis_index("s")
        wid = sid * _NC + lax.axis_index("c")
        base = wid * per
        pltpu.sync_copy(tab_hbm.at[pl.ds(sid * rpt, rpt)],
                        tab_sh.at[pl.ds(sid * rpt, rpt)])
        plsc.subcore_barrier()

        def step(c, carry):
            off = base + c * B
            pltpu.sync_copy(idx_hbm.at[pl.ds(off, B)], idx_v)
            pltpu.async_copy(tab_sh.at[idx_v], rows_v, sem).wait()
            pltpu.sync_copy(rows_v, out_hbm.at[pl.ds(off, B)])
            return carry

        lax.fori_loop(0, nfull, step, 0)
        if tail:
            off = base + nfull * B
            pltpu.sync_copy(idx_hbm.at[pl.ds(off, tail)], idx_t)
            pltpu.async_copy(tab_sh.at[idx_t], rows_t, sem).wait()
            pltpu.sync_copy(rows_t, out_hbm.at[pl.ds(off, tail)])

    return k


def _gather16(table16, idx):
    return _gather16_kernel(idx.shape[0])(table16, idx)


@functools.cache
def _gsg_kernel(E, D, scaled):
    per = E // _NW
    B = min(per, 64)
    nfull, tail = per // B, per % B
    rpt = _NPAD // _NS
    zb = 64
    nz = rpt // zb

    scratch = [
        pltpu.VMEM((B,), jnp.int32),
        pltpu.VMEM((B,), jnp.int32),
        pltpu.VMEM((B, D), jnp.float32),
        pltpu.VMEM((B, D), jnp.float32),
        pltpu.VMEM((max(tail, 8),), jnp.int32),
        pltpu.VMEM((max(tail, 8),), jnp.int32),
        pltpu.VMEM((max(tail, 8), D), jnp.float32),
        pltpu.VMEM((B, 16), jnp.float32),
        pltpu.VMEM((zb, D), jnp.float32),
        pltpu.VMEM_SHARED((_NPAD, D), jnp.float32),
        pltpu.SemaphoreType.DMA,
    ]

    def body(table_hbm, src_hbm, dst_hbm, scale_hbm, out_hbm,
             sidx_v, didx_v, rows_v, rows2_v, sidx_t, didx_t, rows_t, sc_v,
             zero_v, acc_sh, sem):
        cid = lax.axis_index("c")
        sid = lax.axis_index("s")
        wid = sid * _NC + cid
        base = wid * per

        zv = jnp.zeros((16,), jnp.float32)

        def zrow(r, carry):
            def zcol(cc, carry2):
                zero_v[r, pl.ds(cc * 16, 16)] = zv
                return carry2
            return lax.fori_loop(0, D // 16, zcol, carry)

        lax.fori_loop(0, zb, zrow, 0)
        for z in range(nz):
            pltpu.sync_copy(zero_v,
                            acc_sh.at[pl.ds(sid * rpt + z * zb, zb)])
        plsc.subcore_barrier()

        def do_chunk(off, m, si, di, rows):
            pltpu.sync_copy(src_hbm.at[pl.ds(off, m)], si)
            pltpu.async_copy(table_hbm.at[si], rows, sem).wait()
            out_rows = rows
            if scaled:
                pltpu.sync_copy(scale_hbm.at[pl.ds(off, m)],
                                sc_v.at[pl.ds(0, m)])

                def scale_row(r, carry):
                    s16 = sc_v[r, :]

                    def scol(cc, carry2):
                        rows2_v[r, pl.ds(cc * 16, 16)] = \
                            rows[r, pl.ds(cc * 16, 16)] * s16
                        return carry2
                    return lax.fori_loop(0, D // 16, scol, carry)

                lax.fori_loop(0, m, scale_row, 0)
                out_rows = rows2_v if m == B else rows2_v.at[pl.ds(0, m)]
            pltpu.sync_copy(dst_hbm.at[pl.ds(off, m)], di)
            pltpu.sync_copy(out_rows, acc_sh.at[di], add=True)

        def step(c, carry):
            do_chunk(base + c * B, B, sidx_v, didx_v, rows_v)
            return carry

        lax.fori_loop(0, nfull, step, 0)
        if tail:
            do_chunk(base + nfull * B, tail, sidx_t, didx_t, rows_t)
        plsc.subcore_barrier()

        for z in range(nz):
            r0 = sid * rpt + z * zb
            pltpu.sync_copy(acc_sh.at[pl.ds(r0, zb)],
                            out_hbm.at[cid, pl.ds(r0, zb)])

    kw = dict(
        out_type=jax.ShapeDtypeStruct((_NC, _NPAD, D), jnp.float32),
        mesh=_sc_mesh(),
        scratch_types=scratch,
    )
    if scaled:
        @functools.partial(pl.kernel, **kw)
        def k(table_hbm, src_hbm, dst_hbm, scale_hbm, out_hbm, *scr):
            body(table_hbm, src_hbm, dst_hbm, scale_hbm, out_hbm, *scr)
    else:
        @functools.partial(pl.kernel, **kw)
        def k(table_hbm, src_hbm, dst_hbm, out_hbm, *scr):
            body(table_hbm, src_hbm, dst_hbm, None, out_hbm, *scr)

    return k


def _gather_scale_scatter(table, src, dst, scale16=None):
    E = src.shape[0]
    D = table.shape[1]
    if scale16 is None:
        return _gsg_kernel(E, D, False)(table, src, dst)
    return _gsg_kernel(E, D, True)(table, src, dst, scale16)


def _proj_body(h_ref, wl_ref, bl_ref, wr_ref, br_ref, la_ref, we_ref,
               att_ref, xl_ref, xr_ref, ews_ref):
    h = h_ref[...]
    xl = jnp.dot(h, wl_ref[...], preferred_element_type=jnp.float32) + bl_ref[...]
    xr = jnp.dot(h, wr_ref[...], preferred_element_type=jnp.float32) + br_ref[...]
    eel = jnp.dot(la_ref[...], we_ref[...], preferred_element_type=jnp.float32)
    m = xl + xr + eel
    m = jnp.where(m > 0, m, NEG_SLOPE * m) * att_ref[...]
    s0 = jnp.sum(m[:, :HIDDEN], axis=1, keepdims=True)
    s1 = jnp.sum(m[:, HIDDEN:], axis=1, keepdims=True)
    e = jnp.exp(jnp.concatenate([s0, s1], axis=1))
    ews_ref[...] = jnp.pad(e, ((0, 0), (0, 14)))
    xl_ref[...] = xl
    xr_ref[...] = xr


def _tc_proj(h, la, p):
    n = h.shape[0]
    grid = n // _RBLK
    din = h.shape[1]
    attv = p['att'].reshape(1, HEADS * HIDDEN)
    return pl.pallas_call(
        _proj_body,
        grid=(grid,),
        in_specs=[
            pl.BlockSpec((_RBLK, din), lambda i: (i, 0)),
            pl.BlockSpec((din, HEADS * HIDDEN), lambda i: (0, 0)),
            pl.BlockSpec((1, HEADS * HIDDEN), lambda i: (0, 0)),
            pl.BlockSpec((din, HEADS * HIDDEN), lambda i: (0, 0)),
            pl.BlockSpec((1, HEADS * HIDDEN), lambda i: (0, 0)),
            pl.BlockSpec((_RBLK, EMB16), lambda i: (i, 0)),
            pl.BlockSpec((EMB16, HEADS * HIDDEN), lambda i: (0, 0)),
            pl.BlockSpec((1, HEADS * HIDDEN), lambda i: (0, 0)),
        ],
        out_specs=[
            pl.BlockSpec((_RBLK, HEADS * HIDDEN), lambda i: (i, 0)),
            pl.BlockSpec((_RBLK, HEADS * HIDDEN), lambda i: (i, 0)),
            pl.BlockSpec((_RBLK, 16), lambda i: (i, 0)),
        ],
        out_shape=[
            jax.ShapeDtypeStruct((n, HEADS * HIDDEN), jnp.float32),
            jax.ShapeDtypeStruct((n, HEADS * HIDDEN), jnp.float32),
            jax.ShapeDtypeStruct((n, 16), jnp.float32),
        ],
    )(h, p['W_l'], p['b_l'].reshape(1, -1), p['W_r'], p['b_r'].reshape(1, -1),
      la, p['W_e'], attv)


EMB16 = 16


def _edge_body(xls_ref, xrd_ref, oh_ref, ev_ref, eemb_ref, we_ref, att_ref,
               ew_ref, eb0_ref, eb1_ref):
    eetab = jnp.dot(eemb_ref[...], we_ref[...],
                    preferred_element_type=jnp.float32)
    ee = jnp.dot(oh_ref[...], eetab, preferred_element_type=jnp.float32)
    m = xls_ref[...] + xrd_ref[...] + ee
    m = jnp.where(m > 0, m, NEG_SLOPE * m) * att_ref[...]
    s0 = jnp.sum(m[:, :HIDDEN], axis=1, keepdims=True)
    s1 = jnp.sum(m[:, HIDDEN:], axis=1, keepdims=True)
    ev = ev_ref[...]
    e0 = jnp.exp(s0) * ev
    e1 = jnp.exp(s1) * ev
    ew_ref[...] = jnp.pad(jnp.concatenate([e0, e1], axis=1),
                          ((0, 0), (0, 14)))
    eb0_ref[...] = jnp.broadcast_to(e0, (e0.shape[0], 16))
    eb1_ref[...] = jnp.broadcast_to(e1, (e1.shape[0], 16))


def _tc_edge(xls, xrd, onehot, ev1, eemb, we, att):
    grid = N_EDGES // _EBLK
    attv = att.reshape(1, HEADS * HIDDEN)
    return pl.pallas_call(
        _edge_body,
        grid=(grid,),
        in_specs=[
            pl.BlockSpec((_EBLK, HEADS * HIDDEN), lambda i: (i, 0)),
            pl.BlockSpec((_EBLK, HEADS * HIDDEN), lambda i: (i, 0)),
            pl.BlockSpec((_EBLK, EMB16), lambda i: (i, 0)),
            pl.BlockSpec((_EBLK, 1), lambda i: (i, 0)),
            pl.BlockSpec((EMB16, EMB16), lambda i: (0, 0)),
            pl.BlockSpec((EMB16, HEADS * HIDDEN), lambda i: (0, 0)),
            pl.BlockSpec((1, HEADS * HIDDEN), lambda i: (0, 0)),
        ],
        out_specs=[
            pl.BlockSpec((_EBLK, 16), lambda i: (i, 0)),
            pl.BlockSpec((_EBLK, 16), lambda i: (i, 0)),
            pl.BlockSpec((_EBLK, 16), lambda i: (i, 0)),
        ],
        out_shape=[
            jax.ShapeDtypeStruct((N_EDGES, 16), jnp.float32),
            jax.ShapeDtypeStruct((N_EDGES, 16), jnp.float32),
            jax.ShapeDtypeStruct((N_EDGES, 16), jnp.float32),
        ],
    )(xls, xrd, onehot, ev1, eemb, we, attv)


def _combine_body(dp_ref, ews_ref, dr_ref):
    d0 = dp_ref[0, :, 0:1] + dp_ref[1, :, 0:1] + ews_ref[:, 0:1]
    d1 = dp_ref[0, :, 1:2] + dp_ref[1, :, 1:2] + ews_ref[:, 1:2]
    r0 = 1.0 / jnp.maximum(d0, 1e-16)
    r1 = 1.0 / jnp.maximum(d1, 1e-16)
    dr_ref[...] = jnp.pad(jnp.concatenate([r0, r1], axis=1),
                          ((0, 0), (0, 14)))


def _tc_combine(dparts, ews):
    n = ews.shape[0]
    grid = n // _RBLK
    return pl.pallas_call(
        _combine_body,
        grid=(grid,),
        in_specs=[
            pl.BlockSpec((2, _RBLK, 16), lambda i: (0, i, 0)),
            pl.BlockSpec((_RBLK, 16), lambda i: (i, 0)),
        ],
        out_specs=pl.BlockSpec((_RBLK, 16), lambda i: (i, 0)),
        out_shape=jax.ShapeDtypeStruct((n, 16), jnp.float32),
    )(dparts, ews)


def _vals_body(xls_ref, ew_ref, v0_ref, v1_ref):
    xls = xls_ref[...]
    v0_ref[...] = xls[:, :HIDDEN] * ew_ref[:, 0:1]
    v1_ref[...] = xls[:, HIDDEN:] * ew_ref[:, 1:2]


def _tc_vals(xls, ew):
    grid = N_EDGES // _EBLK
    return pl.pallas_call(
        _vals_body,
        grid=(grid,),
        in_specs=[
            pl.BlockSpec((_EBLK, HEADS * HIDDEN), lambda i: (i, 0)),
            pl.BlockSpec((_EBLK, 16), lambda i: (i, 0)),
        ],
        out_specs=[
            pl.BlockSpec((_EBLK, HIDDEN), lambda i: (i, 0)),
            pl.BlockSpec((_EBLK, HIDDEN), lambda i: (i, 0)),
        ],
        out_shape=[
            jax.ShapeDtypeStruct((N_EDGES, HIDDEN), jnp.float32),
            jax.ShapeDtypeStruct((N_EDGES, HIDDEN), jnp.float32),
        ],
    )(xls, ew)


def _post_body(p0_ref, p1_ref, xl_ref, ews_ref, dr_ref, b_ref, h_ref):
    xl = xl_ref[...]
    o0 = (p0_ref[0] + p0_ref[1] + xl[:, :HIDDEN] * ews_ref[:, 0:1]) \
        * dr_ref[:, 0:1]
    o1 = (p1_ref[0] + p1_ref[1] + xl[:, HIDDEN:] * ews_ref[:, 1:2]) \
        * dr_ref[:, 1:2]
    h_ref[...] = jnp.maximum(0.5 * (o0 + o1) + b_ref[...], 0.0)


def _tc_post(p0, p1, xl, ews, denomr, bias):
    n = xl.shape[0]
    grid = n // _RBLK
    return pl.pallas_call(
        _post_body,
        grid=(grid,),
        in_specs=[
            pl.BlockSpec((2, _RBLK, HIDDEN), lambda i: (0, i, 0)),
            pl.BlockSpec((2, _RBLK, HIDDEN), lambda i: (0, i, 0)),
            pl.BlockSpec((_RBLK, HEADS * HIDDEN), lambda i: (i, 0)),
            pl.BlockSpec((_RBLK, 16), lambda i: (i, 0)),
            pl.BlockSpec((_RBLK, 16), lambda i: (i, 0)),
            pl.BlockSpec((1, HIDDEN), lambda i: (0, 0)),
        ],
        out_specs=pl.BlockSpec((_RBLK, HIDDEN), lambda i: (i, 0)),
        out_shape=jax.ShapeDtypeStruct((n, HIDDEN), jnp.float32),
    )(p0, p1, xl, ews, denomr, bias.reshape(1, HIDDEN))


def _lookup_body(oh_ref, tab_ref, o_ref):
    o_ref[...] = jnp.dot(oh_ref[...], tab_ref[...],
                         preferred_element_type=jnp.float32)


def _tc_lookup(onehot, tab, blk):
    total, v = onehot.shape
    d = tab.shape[1]
    grid = total // blk
    return pl.pallas_call(
        _lookup_body,
        grid=(grid,),
        in_specs=[
            pl.BlockSpec((blk, v), lambda i: (i, 0)),
            pl.BlockSpec((v, d), lambda i: (0, 0)),
        ],
        out_specs=pl.BlockSpec((blk, d), lambda i: (i, 0)),
        out_shape=jax.ShapeDtypeStruct((total, d), jnp.float32),
    )(onehot, tab)


def _lvals_body(oh_ref, ev_ref, eemb_ref, o_ref):
    ee = jnp.dot(oh_ref[...], eemb_ref[...],
                 preferred_element_type=jnp.float32)
    ev = ev_ref[...]
    o_ref[...] = jnp.pad(jnp.concatenate([ev, ee * ev], axis=1),
                         ((0, 0), (0, 15)))


def _tc_lvals(onehot, ev1, eemb):
    grid = N_EDGES // _EBLK
    return pl.pallas_call(
        _lvals_body,
        grid=(grid,),
        in_specs=[
            pl.BlockSpec((_EBLK, EMB16), lambda i: (i, 0)),
            pl.BlockSpec((_EBLK, 1), lambda i: (i, 0)),
            pl.BlockSpec((EMB16, EMB16), lambda i: (0, 0)),
        ],
        out_specs=pl.BlockSpec((_EBLK, 32), lambda i: (i, 0)),
        out_shape=jax.ShapeDtypeStruct((N_EDGES, 32), jnp.float32),
    )(onehot, ev1, eemb)


def _avals_body(hs_ref, ev_ref, o_ref):
    o_ref[...] = hs_ref[...] * ev_ref[...]


def _tc_avals(hsrc, ev1):
    grid = N_EDGES // _EBLK
    return pl.pallas_call(
        _avals_body,
        grid=(grid,),
        in_specs=[
            pl.BlockSpec((_EBLK, HIDDEN), lambda i: (i, 0)),
            pl.BlockSpec((_EBLK, 1), lambda i: (i, 0)),
        ],
        out_specs=pl.BlockSpec((_EBLK, HIDDEN), lambda i: (i, 0)),
        out_shape=jax.ShapeDtypeStruct((N_EDGES, HIDDEN), jnp.float32),
    )(hsrc, ev1)


def _score_body(ap_ref, h_ref, wrel_ref, wroot_ref, brel_ref, s_ref):
    a = ap_ref[0] + ap_ref[1]
    s = (jnp.sum(a * wrel_ref[...], axis=1, keepdims=True)
         + jnp.sum(h_ref[...] * wroot_ref[...], axis=1, keepdims=True)
         + brel_ref[0:1, 0:1])
    s_ref[...] = jnp.pad(jnp.tanh(s), ((0, 0), (0, 15)))


def _tc_score(aparts, h_out, wrel, wroot, brel):
    n = h_out.shape[0]
    grid = n // _RBLK
    return pl.pallas_call(
        _score_body,
        grid=(grid,),
        in_specs=[
            pl.BlockSpec((2, _RBLK, HIDDEN), lambda i: (0, i, 0)),
            pl.BlockSpec((_RBLK, HIDDEN), lambda i: (i, 0)),
            pl.BlockSpec((1, HIDDEN), lambda i: (0, 0)),
            pl.BlockSpec((1, HIDDEN), lambda i: (0, 0)),
            pl.BlockSpec((1, HIDDEN), lambda i: (0, 0)),
        ],
        out_specs=pl.BlockSpec((_RBLK, 16), lambda i: (i, 0)),
        out_shape=jax.ShapeDtypeStruct((n, 16), jnp.float32),
    )(aparts, h_out, wrel.reshape(1, HIDDEN), wroot.reshape(1, HIDDEN),
      jnp.broadcast_to(brel.reshape(1, 1), (1, HIDDEN)))


@functools.cache
def _topk_kernel(k):
    rows = _NPAD // 128

    def body(s_ref, sel_ref):
        f = s_ref[...]
        u = lax.bitcast_convert_type(f, jnp.uint32)
        sign = u >= jnp.uint32(0x80000000)
        ukey = u ^ jnp.where(sign, jnp.uint32(0xFFFFFFFF),
                             jnp.uint32(0x80000000))

        def count_ge(t):
            return jnp.sum((ukey >= t).astype(jnp.int32))

        def bs1(_, carry):
            lo, hi = carry
            mid = lo + (hi - lo) // jnp.uint32(2)
            c = count_ge(mid)
            big = c >= k
            return (jnp.where(big, mid, lo), jnp.where(big, hi, mid))

        lo, hi = lax.fori_loop(
            0, 33, bs1, (jnp.uint32(0), jnp.uint32(0xFFFFFFFF)))
        v = lo
        c1 = jnp.sum((ukey > v).astype(jnp.int32))
        r = k - c1
        eq = ukey == v
        idx = (lax.broadcasted_iota(jnp.int32, (rows, 128), 0) * 128
               + lax.broadcasted_iota(jnp.int32, (rows, 128), 1))

        def bs2(_, carry):
            lo2, hi2 = carry
            mid = lo2 + (hi2 - lo2) // 2
            c = jnp.sum((eq & (idx <= mid)).astype(jnp.int32))
            ok = c >= r
            return (jnp.where(ok, lo2, mid), jnp.where(ok, mid, hi2))

        lo2, hi2 = lax.fori_loop(0, 15, bs2, (jnp.int32(-1),
                                              jnp.int32(_NPAD - 1)))
        j = hi2
        sel = (ukey > v) | (eq & (idx <= j))
        sel_ref[...] = sel.astype(jnp.float32)

    return pl.pallas_call(
        body,
        out_shape=jax.ShapeDtypeStruct((rows, 128), jnp.float32),
    )


def _topk_mask(smask, k):
    rows = _NPAD // 128
    pad = jnp.full((_NPAD - N_NODES,), -2.0, jnp.float32)
    s80 = jnp.concatenate([smask, pad]).reshape(rows, 128)
    sel = _topk_kernel(k)(s80)
    return sel.reshape(-1)[:N_NODES]


def _xnew_body(h_ref, s_ref, sel_ref, xn_ref, gs_ref, gm_ref):
    pid = pl.program_id(0)
    sc = s_ref[:, 0:1]
    se = sel_ref[:, 0:1]
    xn = h_ref[...] * sc * se
    xn_ref[...] = xn

    @pl.when(pid == 0)
    def _():
        gs_ref[...] = jnp.zeros_like(gs_ref)
        gm_ref[...] = jnp.full_like(gm_ref, NEG_BIG)

    gs_ref[...] += jnp.sum(xn, axis=0, keepdims=True)
    masked = jnp.where(se > 0, xn, NEG_BIG)
    gm_ref[...] = jnp.maximum(gm_ref[...], jnp.max(masked, axis=0,
                                                   keepdims=True))


def _tc_xnew(h_out, score16, sel16):
    n = h_out.shape[0]
    grid = n // _RBLK
    return pl.pallas_call(
        _xnew_body,
        grid=(grid,),
        in_specs=[
            pl.BlockSpec((_RBLK, HIDDEN), lambda i: (i, 0)),
            pl.BlockSpec((_RBLK, 16), lambda i: (i, 0)),
            pl.BlockSpec((_RBLK, 16), lambda i: (i, 0)),
        ],
        out_specs=[
            pl.BlockSpec((_RBLK, HIDDEN), lambda i: (i, 0)),
            pl.BlockSpec((1, HIDDEN), lambda i: (0, 0)),
            pl.BlockSpec((1, HIDDEN), lambda i: (0, 0)),
        ],
        out_shape=[
            jax.ShapeDtypeStruct((n, HIDDEN), jnp.float32),
            jax.ShapeDtypeStruct((1, HIDDEN), jnp.float32),
            jax.ShapeDtypeStruct((1, HIDDEN), jnp.float32),
        ],
    )(h_out, score16, sel16)


def _head_body(x_ref, w1_ref, b1_ref, w2_ref, b2_ref, w3_ref, b3_ref,
               logits_ref, probs_ref):
    x = x_ref[...]
    h1 = jnp.maximum(jnp.dot(x, w1_ref[...],
                             preferred_element_type=jnp.float32)
                     + b1_ref[...], 0.0)
    h2 = jnp.maximum(jnp.dot(h1, w2_ref[...],
                             preferred_element_type=jnp.float32)
                     + b2_ref[...], 0.0)
    logits = (jnp.dot(h2, w3_ref[...], preferred_element_type=jnp.float32)
              + b3_ref[...])
    ncls = lax.broadcasted_iota(jnp.int32, logits.shape, 1) < 2
    lm = jnp.where(ncls, logits, NEG_BIG)
    mx = jnp.max(lm, axis=1, keepdims=True)
    ew = jnp.where(ncls, jnp.exp(lm - mx), 0.0)
    probs_ref[...] = ew / jnp.sum(ew, axis=1, keepdims=True)
    logits_ref[...] = logits


def _mlp_head(out_vec, params):
    x = jnp.zeros((8, 2 * HIDDEN), jnp.float32).at[0].set(out_vec)
    logits, probs = pl.pallas_call(
        _head_body,
        out_shape=(jax.ShapeDtypeStruct((8, 8), jnp.float32),
                   jax.ShapeDtypeStruct((8, 8), jnp.float32)),
    )(x, params['lin1_W'], params['lin1_b'].reshape(1, -1),
      params['lin2_W'], params['lin2_b'].reshape(1, -1),
      jnp.pad(params['lin3_W'], ((0, 0), (0, 6))),
      jnp.pad(params['lin3_b'], (0, 6)).reshape(1, -1))
    return logits[0:1, 0:2], probs[0:1, 0:2]


def kernel(x, edge_index, edge_attr, node_attr, random_walk_pe, batch,
           label, params):
    n = x.shape[0]
    src, dst = edge_index[0], edge_index[1]

    onehot = (edge_attr[:, None] == jnp.arange(EMB16, dtype=edge_attr.dtype)
              ).astype(jnp.float32)
    onehot_n = (node_attr[:, None]
                == jnp.arange(32, dtype=node_attr.dtype)).astype(jnp.float32)
    na_emb = _tc_lookup(onehot_n, params['node_emb'], _RBLK)

    evalid = jnp.ones((N_EDGES,), jnp.float32)
    valid_n = jnp.ones((n,), jnp.float32)
    rwpe = random_walk_pe
    n_cur = n
    layer_embs = []
    for i in range(NUM_LAYERS):
        cp = params['convs'][i]
        pp = params['pools'][i]

        h = jnp.concatenate([x, rwpe, na_emb], axis=1)

        vals32 = _tc_lvals(onehot, evalid[:, None], params['edge_emb'])
        dl = _scatter_add(vals32, dst)
        degloop = (dl[0] + dl[1])[:n]
        deg = degloop[:, 0:1]
        loop_attr = degloop[:, 1:17] / jnp.maximum(deg, 1.0)

        xl, xr, ews = _tc_proj(h, loop_attr, cp)

        xls = _gather_rows(xl, src)
        xrd = _gather_rows(xr, dst)

        ew, ewb0, ewb1 = _tc_edge(xls, xrd, onehot, evalid[:, None],
                                  params['edge_emb'], cp['W_e'], cp['att'])

        dparts = _scatter_add(ew, dst)
        denomr = _tc_combine(dparts[:, :n], ews)

        p0 = _gather_scale_scatter(xl[:, :HIDDEN], src, dst, ewb0)[:, :n]
        p0, src_b, dst_b, ewb1 = lax.optimization_barrier(
            (p0, src, dst, ewb1))
        p1 = _gather_scale_scatter(xl[:, HIDDEN:], src_b, dst_b, ewb1)[:, :n]
        h_out = _tc_post(p0, p1, xl, ews, denomr, cp['bias'])

        if i == 0:
            aparts = _gather_scale_scatter(h_out, src, dst)[:, :n]
        else:
            aparts = _gather_scale_scatter(h_out, src, dst, evalid16)[:, :n]
        score16 = _tc_score(aparts, h_out, pp['W_rel'], pp['W_root'],
                            pp['b_rel'])

        k = int(math.ceil(RATIO * n_cur))
        smask = jnp.where(valid_n > 0, score16[:, 0], -2.0)
        sel = _topk_mask(smask, k)

        sel16 = jnp.broadcast_to(sel[:, None], (n, 16))
        x, gsum, gmax = _tc_xnew(h_out, score16, sel16)
        gmean = gsum / float(k)
        layer_embs.append(jnp.concatenate([gmean, gmax], axis=1))

        if i + 1 < NUM_LAYERS:
            selp = jnp.pad(jnp.broadcast_to(sel[:, None], (n, 16)),
                           ((0, _NPAD - n), (0, 0)))
            ssrc = _gather16(selp, src)
            sdst = _gather16(selp, dst)
            evalid16 = ssrc * sdst * evalid[:, None]
            evalid = evalid16[:, 0]
        valid_n = sel
        n_cur = k

    out = (layer_embs[0] + layer_embs[1])[0]
    logits, probs = _mlp_head(out, params)
    return (logits, probs, label)


# speedup vs baseline: 1.2579x; 1.2579x over previous
import functools
import math

import jax
import jax.numpy as jnp
from jax import lax
from jax.experimental import pallas as pl
from jax.experimental.pallas import tpu as pltpu
from jax.experimental.pallas import tpu_sc as plsc

N_NODES = 10000
N_EDGES = 160000
HIDDEN = 128
HEADS = 2
NUM_LAYERS = 2
RATIO = 0.5
NEG_SLOPE = 0.2
NEG_BIG = -1e30

_NC, _NS = 2, 16
_NW = _NC * _NS
_NPAD = 10240
_RBLK = 1000
_EBLK = 2000

_sc_mesh = functools.partial(
    plsc.VectorSubcoreMesh, core_axis_name="c", subcore_axis_name="s")


def _split_chunks(total):
    per = total // _NW
    b = min(per, 128)
    return per, b, per // b, per % b


@functools.cache
def _gather_rows_kernel(E, D):
    per, B, nfull, tail = _split_chunks(E)

    scratch = [
        pltpu.VMEM((B,), jnp.int32),
        pltpu.VMEM((B, D), jnp.float32),
        pltpu.VMEM((max(tail, 8),), jnp.int32),
        pltpu.VMEM((max(tail, 8), D), jnp.float32),
        pltpu.SemaphoreType.DMA,
    ]

    @functools.partial(
        pl.kernel,
        out_type=jax.ShapeDtypeStruct((E, D), jnp.float32),
        mesh=_sc_mesh(),
        scratch_types=scratch,
    )
    def k(table_hbm, idx_hbm, out_hbm, idx_v, rows_v, idx_t, rows_t, sem):
        wid = lax.axis_index("s") * _NC + lax.axis_index("c")
        base = wid * per

        def step(c, carry):
            off = base + c * B
            pltpu.sync_copy(idx_hbm.at[pl.ds(off, B)], idx_v)
            pltpu.async_copy(table_hbm.at[idx_v], rows_v, sem).wait()
            pltpu.sync_copy(rows_v, out_hbm.at[pl.ds(off, B)])
            return carry

        lax.fori_loop(0, nfull, step, 0)
        if tail:
            off = base + nfull * B
            pltpu.sync_copy(idx_hbm.at[pl.ds(off, tail)], idx_t)
            pltpu.async_copy(table_hbm.at[idx_t], rows_t, sem).wait()
            pltpu.sync_copy(rows_t, out_hbm.at[pl.ds(off, tail)])

    return k


def _gather_rows(table, idx):
    E = idx.shape[0]
    return _gather_rows_kernel(E, table.shape[1])(table, idx)


@functools.cache
def _scatter_add_kernel(E, D):
    per, B, nfull, tail = _split_chunks(E)
    rpt = _NPAD // _NS
    zb = 128
    nz = rpt // zb

    scratch = [
        pltpu.VMEM((B,), jnp.int32),
        pltpu.VMEM((B, D), jnp.float32),
        pltpu.VMEM((max(tail, 8),), jnp.int32),
        pltpu.VMEM((max(tail, 8), D), jnp.float32),
        pltpu.VMEM((zb, D), jnp.float32),
        pltpu.VMEM_SHARED((_NPAD, D), jnp.float32),
        pltpu.SemaphoreType.DMA,
    ]
    assert tail in (0, max(tail, 8))

    @functools.partial(
        pl.kernel,
        out_type=jax.ShapeDtypeStruct((_NC, _NPAD, D), jnp.float32),
        mesh=_sc_mesh(),
        scratch_types=scratch,
    )
    def k(vals_hbm, idx_hbm, out_hbm, idx_v, rows_v, idx_t, rows_t,
          zero_v, acc_sh, sem):
        cid = lax.axis_index("c")
        sid = lax.axis_index("s")
        wid = sid * _NC + cid
        base = wid * per

        zv = jnp.zeros((16,), jnp.float32)

        def zrow(r, carry):
            def zcol(cc, carry2):
                zero_v[r, pl.ds(cc * 16, 16)] = zv
                return carry2
            return lax.fori_loop(0, D // 16, zcol, carry)

        lax.fori_loop(0, zb, zrow, 0)
        for z in range(nz):
            pltpu.sync_copy(zero_v,
                            acc_sh.at[pl.ds(sid * rpt + z * zb, zb)])
        plsc.subcore_barrier()

        def step(c, carry):
            off = base + c * B
            pltpu.sync_copy(idx_hbm.at[pl.ds(off, B)], idx_v)
            pltpu.sync_copy(vals_hbm.at[pl.ds(off, B)], rows_v)
            pltpu.sync_copy(rows_v, acc_sh.at[idx_v], add=True)
            return carry

        lax.fori_loop(0, nfull, step, 0)
        if tail:
            off = base + nfull * B
            pltpu.sync_copy(idx_hbm.at[pl.ds(off, tail)], idx_t)
            pltpu.sync_copy(vals_hbm.at[pl.ds(off, tail)], rows_t)
            pltpu.sync_copy(rows_t, acc_sh.at[idx_t], add=True)
        plsc.subcore_barrier()

        for z in range(nz):
            r0 = sid * rpt + z * zb
            pltpu.sync_copy(acc_sh.at[pl.ds(r0, zb)],
                            out_hbm.at[cid, pl.ds(r0, zb)])

    return k


def _scatter_add(vals, idx):
    parts = _scatter_add_kernel(idx.shape[0], vals.shape[1])(vals, idx)
    return parts


@functools.cache
def _gather16_kernel(E):
    per, B, nfull, tail = _split_chunks(E)
    rpt = _NPAD // _NS

    scratch = [
        pltpu.VMEM((B,), jnp.int32),
        pltpu.VMEM((B, 16), jnp.float32),
        pltpu.VMEM((max(tail, 8),), jnp.int32),
        pltpu.VMEM((max(tail, 8), 16), jnp.float32),
        pltpu.VMEM_SHARED((_NPAD, 16), jnp.float32),
        pltpu.SemaphoreType.DMA,
    ]

    @functools.partial(
        pl.kernel,
        out_type=jax.ShapeDtypeStruct((E, 16), jnp.float32),
        mesh=_sc_mesh(),
        scratch_types=scratch,
    )
    def k(tab_hbm, idx_hbm, out_hbm, idx_v, rows_v, idx_t, rows_t,
          tab_sh, sem):
        sid = lax.axis_index("s")
        wid = sid * _NC + lax.axis_index("c")
        base = wid * per
        pltpu.sync_copy(tab_hbm.at[pl.ds(sid * rpt, rpt)],
                        tab_sh.at[pl.ds(sid * rpt, rpt)])
        plsc.subcore_barrier()

        def step(c, carry):
            off = base + c * B
            pltpu.sync_copy(idx_hbm.at[pl.ds(off, B)], idx_v)
            pltpu.async_copy(tab_sh.at[idx_v], rows_v, sem).wait()
            pltpu.sync_copy(rows_v, out_hbm.at[pl.ds(off, B)])
            return carry

        lax.fori_loop(0, nfull, step, 0)
        if tail:
            off = base + nfull * B
            pltpu.sync_copy(idx_hbm.at[pl.ds(off, tail)], idx_t)
            pltpu.async_copy(tab_sh.at[idx_t], rows_t, sem).wait()
            pltpu.sync_copy(rows_t, out_hbm.at[pl.ds(off, tail)])

    return k


def _gather16(table16, idx):
    return _gather16_kernel(idx.shape[0])(table16, idx)


@functools.cache
def _gsg_plain_kernel(E, D):
    per, B, nfull, tail = _split_chunks(E)
    rpt = _NPAD // _NS
    zb = 128
    nz = rpt // zb

    scratch = [
        pltpu.VMEM((B,), jnp.int32),
        pltpu.VMEM((B,), jnp.int32),
        pltpu.VMEM((B, D), jnp.float32),
        pltpu.VMEM((max(tail, 8),), jnp.int32),
        pltpu.VMEM((max(tail, 8),), jnp.int32),
        pltpu.VMEM((max(tail, 8), D), jnp.float32),
        pltpu.VMEM((zb, D), jnp.float32),
        pltpu.VMEM_SHARED((_NPAD, D), jnp.float32),
        pltpu.SemaphoreType.DMA,
    ]

    @functools.partial(
        pl.kernel,
        out_type=jax.ShapeDtypeStruct((_NC, _NPAD, D), jnp.float32),
        mesh=_sc_mesh(),
        scratch_types=scratch,
    )
    def k(table_hbm, src_hbm, dst_hbm, out_hbm, sidx_v, didx_v, rows_v,
          sidx_t, didx_t, rows_t, zero_v, acc_sh, sem):
        cid = lax.axis_index("c")
        sid = lax.axis_index("s")
        wid = sid * _NC + cid
        base = wid * per

        zv = jnp.zeros((16,), jnp.float32)

        def zrow(r, carry):
            def zcol(cc, carry2):
                zero_v[r, pl.ds(cc * 16, 16)] = zv
                return carry2
            return lax.fori_loop(0, D // 16, zcol, carry)

        lax.fori_loop(0, zb, zrow, 0)
        for z in range(nz):
            pltpu.sync_copy(zero_v,
                            acc_sh.at[pl.ds(sid * rpt + z * zb, zb)])
        plsc.subcore_barrier()

        def step(c, carry):
            off = base + c * B
            pltpu.sync_copy(src_hbm.at[pl.ds(off, B)], sidx_v)
            pltpu.async_copy(table_hbm.at[sidx_v], rows_v, sem).wait()
            pltpu.sync_copy(dst_hbm.at[pl.ds(off, B)], didx_v)
            pltpu.sync_copy(rows_v, acc_sh.at[didx_v], add=True)
            return carry

        lax.fori_loop(0, nfull, step, 0)
        if tail:
            off = base + nfull * B
            pltpu.sync_copy(src_hbm.at[pl.ds(off, tail)], sidx_t)
            pltpu.async_copy(table_hbm.at[sidx_t], rows_t, sem).wait()
            pltpu.sync_copy(dst_hbm.at[pl.ds(off, tail)], didx_t)
            pltpu.sync_copy(rows_t, acc_sh.at[didx_t], add=True)
        plsc.subcore_barrier()

        for z in range(nz):
            r0 = sid * rpt + z * zb
            pltpu.sync_copy(acc_sh.at[pl.ds(r0, zb)],
                            out_hbm.at[cid, pl.ds(r0, zb)])

    return k


def _gather_scatter(table, src, dst):
    return _gsg_plain_kernel(src.shape[0], table.shape[1])(table, src, dst)


def _proj_body(h_ref, wl_ref, bl_ref, wr_ref, br_ref, la_ref, we_ref,
               att_ref, xl_ref, xr_ref, ews_ref):
    h = h_ref[...]
    xl = jnp.dot(h, wl_ref[...], preferred_element_type=jnp.float32) + bl_ref[...]
    xr = jnp.dot(h, wr_ref[...], preferred_element_type=jnp.float32) + br_ref[...]
    eel = jnp.dot(la_ref[...], we_ref[...], preferred_element_type=jnp.float32)
    m = xl + xr + eel
    m = jnp.where(m > 0, m, NEG_SLOPE * m) * att_ref[...]
    s0 = jnp.sum(m[:, :HIDDEN], axis=1, keepdims=True)
    s1 = jnp.sum(m[:, HIDDEN:], axis=1, keepdims=True)
    e = jnp.exp(jnp.concatenate([s0, s1], axis=1))
    ews_ref[...] = jnp.pad(e, ((0, 0), (0, 14)))
    xl_ref[...] = xl
    xr_ref[...] = xr


def _tc_proj(h, la, p):
    n = h.shape[0]
    grid = n // _RBLK
    din = h.shape[1]
    attv = p['att'].reshape(1, HEADS * HIDDEN)
    return pl.pallas_call(
        _proj_body,
        grid=(grid,),
        in_specs=[
            pl.BlockSpec((_RBLK, din), lambda i: (i, 0)),
            pl.BlockSpec((din, HEADS * HIDDEN), lambda i: (0, 0)),
            pl.BlockSpec((1, HEADS * HIDDEN), lambda i: (0, 0)),
            pl.BlockSpec((din, HEADS * HIDDEN), lambda i: (0, 0)),
            pl.BlockSpec((1, HEADS * HIDDEN), lambda i: (0, 0)),
            pl.BlockSpec((_RBLK, EMB16), lambda i: (i, 0)),
            pl.BlockSpec((EMB16, HEADS * HIDDEN), lambda i: (0, 0)),
            pl.BlockSpec((1, HEADS * HIDDEN), lambda i: (0, 0)),
        ],
        out_specs=[
            pl.BlockSpec((_RBLK, HEADS * HIDDEN), lambda i: (i, 0)),
            pl.BlockSpec((_RBLK, HEADS * HIDDEN), lambda i: (i, 0)),
            pl.BlockSpec((_RBLK, 16), lambda i: (i, 0)),
        ],
        out_shape=[
            jax.ShapeDtypeStruct((n, HEADS * HIDDEN), jnp.float32),
            jax.ShapeDtypeStruct((n, HEADS * HIDDEN), jnp.float32),
            jax.ShapeDtypeStruct((n, 16), jnp.float32),
        ],
    )(h, p['W_l'], p['b_l'].reshape(1, -1), p['W_r'], p['b_r'].reshape(1, -1),
      la, p['W_e'], attv)


EMB16 = 16


def _edge_body(xls_ref, xrd_ref, oh_ref, ev_ref, eemb_ref, we_ref, att_ref,
               ew_ref):
    eetab = jnp.dot(eemb_ref[...], we_ref[...],
                    preferred_element_type=jnp.float32)
    ee = jnp.dot(oh_ref[...], eetab, preferred_element_type=jnp.float32)
    m = xls_ref[...] + xrd_ref[...] + ee
    m = jnp.where(m > 0, m, NEG_SLOPE * m) * att_ref[...]
    s0 = jnp.sum(m[:, :HIDDEN], axis=1, keepdims=True)
    s1 = jnp.sum(m[:, HIDDEN:], axis=1, keepdims=True)
    ev = ev_ref[...]
    e0 = jnp.exp(s0) * ev
    e1 = jnp.exp(s1) * ev
    ew_ref[...] = jnp.pad(jnp.concatenate([e0, e1], axis=1),
                          ((0, 0), (0, 14)))


def _tc_edge(xls, xrd, onehot, ev1, eemb, we, att):
    grid = N_EDGES // _EBLK
    attv = att.reshape(1, HEADS * HIDDEN)
    return pl.pallas_call(
        _edge_body,
        grid=(grid,),
        in_specs=[
            pl.BlockSpec((_EBLK, HEADS * HIDDEN), lambda i: (i, 0)),
            pl.BlockSpec((_EBLK, HEADS * HIDDEN), lambda i: (i, 0)),
            pl.BlockSpec((_EBLK, EMB16), lambda i: (i, 0)),
            pl.BlockSpec((_EBLK, 1), lambda i: (i, 0)),
            pl.BlockSpec((EMB16, EMB16), lambda i: (0, 0)),
            pl.BlockSpec((EMB16, HEADS * HIDDEN), lambda i: (0, 0)),
            pl.BlockSpec((1, HEADS * HIDDEN), lambda i: (0, 0)),
        ],
        out_specs=pl.BlockSpec((_EBLK, 16), lambda i: (i, 0)),
        out_shape=jax.ShapeDtypeStruct((N_EDGES, 16), jnp.float32),
    )(xls, xrd, onehot, ev1, eemb, we, attv)


def _combine_body(dp_ref, ews_ref, dr_ref):
    d0 = dp_ref[0, :, 0:1] + dp_ref[1, :, 0:1] + ews_ref[:, 0:1]
    d1 = dp_ref[0, :, 1:2] + dp_ref[1, :, 1:2] + ews_ref[:, 1:2]
    r0 = 1.0 / jnp.maximum(d0, 1e-16)
    r1 = 1.0 / jnp.maximum(d1, 1e-16)
    dr_ref[...] = jnp.pad(jnp.concatenate([r0, r1], axis=1),
                          ((0, 0), (0, 14)))


def _tc_combine(dparts, ews):
    n = ews.shape[0]
    grid = n // _RBLK
    return pl.pallas_call(
        _combine_body,
        grid=(grid,),
        in_specs=[
            pl.BlockSpec((2, _RBLK, 16), lambda i: (0, i, 0)),
            pl.BlockSpec((_RBLK, 16), lambda i: (i, 0)),
        ],
        out_specs=pl.BlockSpec((_RBLK, 16), lambda i: (i, 0)),
        out_shape=jax.ShapeDtypeStruct((n, 16), jnp.float32),
    )(dparts, ews)


def _vals_body(xls_ref, ew_ref, v0_ref, v1_ref):
    xls = xls_ref[...]
    v0_ref[...] = xls[:, :HIDDEN] * ew_ref[:, 0:1]
    v1_ref[...] = xls[:, HIDDEN:] * ew_ref[:, 1:2]


def _tc_vals(xls, ew):
    grid = N_EDGES // _EBLK
    return pl.pallas_call(
        _vals_body,
        grid=(grid,),
        in_specs=[
            pl.BlockSpec((_EBLK, HEADS * HIDDEN), lambda i: (i, 0)),
            pl.BlockSpec((_EBLK, 16), lambda i: (i, 0)),
        ],
        out_specs=[
            pl.BlockSpec((_EBLK, HIDDEN), lambda i: (i, 0)),
            pl.BlockSpec((_EBLK, HIDDEN), lambda i: (i, 0)),
        ],
        out_shape=[
            jax.ShapeDtypeStruct((N_EDGES, HIDDEN), jnp.float32),
            jax.ShapeDtypeStruct((N_EDGES, HIDDEN), jnp.float32),
        ],
    )(xls, ew)


def _post_body(p0_ref, p1_ref, xl_ref, ews_ref, dr_ref, b_ref, vm_ref,
               h_ref, hm_ref):
    xl = xl_ref[...]
    o0 = (p0_ref[0] + p0_ref[1] + xl[:, :HIDDEN] * ews_ref[:, 0:1]) \
        * dr_ref[:, 0:1]
    o1 = (p1_ref[0] + p1_ref[1] + xl[:, HIDDEN:] * ews_ref[:, 1:2]) \
        * dr_ref[:, 1:2]
    h = jnp.maximum(0.5 * (o0 + o1) + b_ref[...], 0.0)
    h_ref[...] = h
    hm_ref[...] = h * vm_ref[:, 0:1]


def _tc_post(p0, p1, xl, ews, denomr, bias, vmask16):
    n = xl.shape[0]
    grid = n // _RBLK
    return pl.pallas_call(
        _post_body,
        grid=(grid,),
        in_specs=[
            pl.BlockSpec((2, _RBLK, HIDDEN), lambda i: (0, i, 0)),
            pl.BlockSpec((2, _RBLK, HIDDEN), lambda i: (0, i, 0)),
            pl.BlockSpec((_RBLK, HEADS * HIDDEN), lambda i: (i, 0)),
            pl.BlockSpec((_RBLK, 16), lambda i: (i, 0)),
            pl.BlockSpec((_RBLK, 16), lambda i: (i, 0)),
            pl.BlockSpec((1, HIDDEN), lambda i: (0, 0)),
            pl.BlockSpec((_RBLK, 16), lambda i: (i, 0)),
        ],
        out_specs=[
            pl.BlockSpec((_RBLK, HIDDEN), lambda i: (i, 0)),
            pl.BlockSpec((_RBLK, HIDDEN), lambda i: (i, 0)),
        ],
        out_shape=[
            jax.ShapeDtypeStruct((n, HIDDEN), jnp.float32),
            jax.ShapeDtypeStruct((n, HIDDEN), jnp.float32),
        ],
    )(p0, p1, xl, ews, denomr, bias.reshape(1, HIDDEN), vmask16)


def _lookup_body(oh_ref, tab_ref, o_ref):
    o_ref[...] = jnp.dot(oh_ref[...], tab_ref[...],
                         preferred_element_type=jnp.float32)


def _tc_lookup(onehot, tab, blk):
    total, v = onehot.shape
    d = tab.shape[1]
    grid = total // blk
    return pl.pallas_call(
        _lookup_body,
        grid=(grid,),
        in_specs=[
            pl.BlockSpec((blk, v), lambda i: (i, 0)),
            pl.BlockSpec((v, d), lambda i: (0, 0)),
        ],
        out_specs=pl.BlockSpec((blk, d), lambda i: (i, 0)),
        out_shape=jax.ShapeDtypeStruct((total, d), jnp.float32),
    )(onehot, tab)


def _lvals_body(oh_ref, ev_ref, eemb_ref, o_ref):
    ee = jnp.dot(oh_ref[...], eemb_ref[...],
                 preferred_element_type=jnp.float32)
    ev = ev_ref[...]
    o_ref[...] = jnp.pad(jnp.concatenate([ev, ee * ev], axis=1),
                         ((0, 0), (0, 15)))


def _tc_lvals(onehot, ev1, eemb):
    grid = N_EDGES // _EBLK
    return pl.pallas_call(
        _lvals_body,
        grid=(grid,),
        in_specs=[
            pl.BlockSpec((_EBLK, EMB16), lambda i: (i, 0)),
            pl.BlockSpec((_EBLK, 1), lambda i: (i, 0)),
            pl.BlockSpec((EMB16, EMB16), lambda i: (0, 0)),
        ],
        out_specs=pl.BlockSpec((_EBLK, 32), lambda i: (i, 0)),
        out_shape=jax.ShapeDtypeStruct((N_EDGES, 32), jnp.float32),
    )(onehot, ev1, eemb)


def _score_body(ap_ref, h_ref, wrel_ref, wroot_ref, brel_ref, s_ref):
    a = ap_ref[0] + ap_ref[1]
    s = (jnp.sum(a * wrel_ref[...], axis=1, keepdims=True)
         + jnp.sum(h_ref[...] * wroot_ref[...], axis=1, keepdims=True)
         + brel_ref[0:1, 0:1])
    s_ref[...] = jnp.pad(jnp.tanh(s), ((0, 0), (0, 15)))


def _tc_score(aparts, h_out, wrel, wroot, brel):
    n = h_out.shape[0]
    grid = n // _RBLK
    return pl.pallas_call(
        _score_body,
        grid=(grid,),
        in_specs=[
            pl.BlockSpec((2, _RBLK, HIDDEN), lambda i: (0, i, 0)),
            pl.BlockSpec((_RBLK, HIDDEN), lambda i: (i, 0)),
            pl.BlockSpec((1, HIDDEN), lambda i: (0, 0)),
            pl.BlockSpec((1, HIDDEN), lambda i: (0, 0)),
            pl.BlockSpec((1, HIDDEN), lambda i: (0, 0)),
        ],
        out_specs=pl.BlockSpec((_RBLK, 16), lambda i: (i, 0)),
        out_shape=jax.ShapeDtypeStruct((n, 16), jnp.float32),
    )(aparts, h_out, wrel.reshape(1, HIDDEN), wroot.reshape(1, HIDDEN),
      jnp.broadcast_to(brel.reshape(1, 1), (1, HIDDEN)))


@functools.cache
def _topk_kernel(k):
    rows = _NPAD // 128

    def body(s_ref, sel_ref):
        f = s_ref[...]
        u = lax.bitcast_convert_type(f, jnp.uint32)
        sign = u >= jnp.uint32(0x80000000)
        ukey = u ^ jnp.where(sign, jnp.uint32(0xFFFFFFFF),
                             jnp.uint32(0x80000000))

        def count_ge(t):
            return jnp.sum((ukey >= t).astype(jnp.int32))

        def bs1(_, carry):
            lo, hi = carry
            mid = lo + (hi - lo) // jnp.uint32(2)
            c = count_ge(mid)
            big = c >= k
            return (jnp.where(big, mid, lo), jnp.where(big, hi, mid))

        lo, hi = lax.fori_loop(
            0, 33, bs1, (jnp.uint32(0), jnp.uint32(0xFFFFFFFF)))
        v = lo
        c1 = jnp.sum((ukey > v).astype(jnp.int32))
        r = k - c1
        eq = ukey == v
        idx = (lax.broadcasted_iota(jnp.int32, (rows, 128), 0) * 128
               + lax.broadcasted_iota(jnp.int32, (rows, 128), 1))

        def bs2(_, carry):
            lo2, hi2 = carry
            mid = lo2 + (hi2 - lo2) // 2
            c = jnp.sum((eq & (idx <= mid)).astype(jnp.int32))
            ok = c >= r
            return (jnp.where(ok, lo2, mid), jnp.where(ok, mid, hi2))

        lo2, hi2 = lax.fori_loop(0, 15, bs2, (jnp.int32(-1),
                                              jnp.int32(_NPAD - 1)))
        j = hi2
        sel = (ukey > v) | (eq & (idx <= j))
        sel_ref[...] = sel.astype(jnp.float32)

    return pl.pallas_call(
        body,
        out_shape=jax.ShapeDtypeStruct((rows, 128), jnp.float32),
    )


def _topk_mask(smask, k):
    rows = _NPAD // 128
    pad = jnp.full((_NPAD - N_NODES,), -2.0, jnp.float32)
    s80 = jnp.concatenate([smask, pad]).reshape(rows, 128)
    sel = _topk_kernel(k)(s80)
    return sel.reshape(-1)[:N_NODES]


def _xnew_body(h_ref, s_ref, sel_ref, xn_ref, gs_ref, gm_ref):
    pid = pl.program_id(0)
    sc = s_ref[:, 0:1]
    se = sel_ref[:, 0:1]
    xn = h_ref[...] * sc * se
    xn_ref[...] = xn

    @pl.when(pid == 0)
    def _():
        gs_ref[...] = jnp.zeros_like(gs_ref)
        gm_ref[...] = jnp.full_like(gm_ref, NEG_BIG)

    gs_ref[...] += jnp.sum(xn, axis=0, keepdims=True)
    masked = jnp.where(se > 0, xn, NEG_BIG)
    gm_ref[...] = jnp.maximum(gm_ref[...], jnp.max(masked, axis=0,
                                                   keepdims=True))


def _tc_xnew(h_out, score16, sel16):
    n = h_out.shape[0]
    grid = n // _RBLK
    return pl.pallas_call(
        _xnew_body,
        grid=(grid,),
        in_specs=[
            pl.BlockSpec((_RBLK, HIDDEN), lambda i: (i, 0)),
            pl.BlockSpec((_RBLK, 16), lambda i: (i, 0)),
            pl.BlockSpec((_RBLK, 16), lambda i: (i, 0)),
        ],
        out_specs=[
            pl.BlockSpec((_RBLK, HIDDEN), lambda i: (i, 0)),
            pl.BlockSpec((1, HIDDEN), lambda i: (0, 0)),
            pl.BlockSpec((1, HIDDEN), lambda i: (0, 0)),
        ],
        out_shape=[
            jax.ShapeDtypeStruct((n, HIDDEN), jnp.float32),
            jax.ShapeDtypeStruct((1, HIDDEN), jnp.float32),
            jax.ShapeDtypeStruct((1, HIDDEN), jnp.float32),
        ],
    )(h_out, score16, sel16)


def _head_body(x_ref, w1_ref, b1_ref, w2_ref, b2_ref, w3_ref, b3_ref,
               logits_ref, probs_ref):
    x = x_ref[...]
    h1 = jnp.maximum(jnp.dot(x, w1_ref[...],
                             preferred_element_type=jnp.float32)
                     + b1_ref[...], 0.0)
    h2 = jnp.maximum(jnp.dot(h1, w2_ref[...],
                             preferred_element_type=jnp.float32)
                     + b2_ref[...], 0.0)
    logits = (jnp.dot(h2, w3_ref[...], preferred_element_type=jnp.float32)
              + b3_ref[...])
    ncls = lax.broadcasted_iota(jnp.int32, logits.shape, 1) < 2
    lm = jnp.where(ncls, logits, NEG_BIG)
    mx = jnp.max(lm, axis=1, keepdims=True)
    ew = jnp.where(ncls, jnp.exp(lm - mx), 0.0)
    probs_ref[...] = ew / jnp.sum(ew, axis=1, keepdims=True)
    logits_ref[...] = logits


def _mlp_head(out_vec, params):
    x = jnp.zeros((8, 2 * HIDDEN), jnp.float32).at[0].set(out_vec)
    logits, probs = pl.pallas_call(
        _head_body,
        out_shape=(jax.ShapeDtypeStruct((8, 8), jnp.float32),
                   jax.ShapeDtypeStruct((8, 8), jnp.float32)),
    )(x, params['lin1_W'], params['lin1_b'].reshape(1, -1),
      params['lin2_W'], params['lin2_b'].reshape(1, -1),
      jnp.pad(params['lin3_W'], ((0, 0), (0, 6))),
      jnp.pad(params['lin3_b'], (0, 6)).reshape(1, -1))
    return logits[0:1, 0:2], probs[0:1, 0:2]


def kernel(x, edge_index, edge_attr, node_attr, random_walk_pe, batch,
           label, params):
    n = x.shape[0]
    src, dst = edge_index[0], edge_index[1]

    onehot = (edge_attr[:, None] == jnp.arange(EMB16, dtype=edge_attr.dtype)
              ).astype(jnp.float32)
    onehot_n = (node_attr[:, None]
                == jnp.arange(32, dtype=node_attr.dtype)).astype(jnp.float32)
    na_emb = _tc_lookup(onehot_n, params['node_emb'], _RBLK)

    evalid = jnp.ones((N_EDGES,), jnp.float32)
    valid_n = jnp.ones((n,), jnp.float32)
    rwpe = random_walk_pe
    n_cur = n
    layer_embs = []
    for i in range(NUM_LAYERS):
        cp = params['convs'][i]
        pp = params['pools'][i]

        h = jnp.concatenate([x, rwpe, na_emb], axis=1)

        vals32 = _tc_lvals(onehot, evalid[:, None], params['edge_emb'])
        dl = _scatter_add(vals32, dst)
        degloop = (dl[0] + dl[1])[:n]
        deg = degloop[:, 0:1]
        loop_attr = degloop[:, 1:17] / jnp.maximum(deg, 1.0)

        xl, xr, ews = _tc_proj(h, loop_attr, cp)

        xls = _gather_rows(xl, src)
        xrd = _gather_rows(xr, dst)

        ew = _tc_edge(xls, xrd, onehot, evalid[:, None],
                      params['edge_emb'], cp['W_e'], cp['att'])

        dparts = _scatter_add(ew, dst)
        denomr = _tc_combine(dparts[:, :n], ews)

        v0, v1 = _tc_vals(xls, ew)
        p0 = _scatter_add(v0, dst)[:, :n]
        p1 = _scatter_add(v1, dst)[:, :n]
        vmask16 = jnp.broadcast_to(valid_n[:, None], (n, 16))
        h_out, hm = _tc_post(p0, p1, xl, ews, denomr, cp['bias'], vmask16)

        aparts = _gather_scatter(hm, src, dst)[:, :n]
        score16 = _tc_score(aparts, h_out, pp['W_rel'], pp['W_root'],
                            pp['b_rel'])

        k = int(math.ceil(RATIO * n_cur))
        smask = jnp.where(valid_n > 0, score16[:, 0], -2.0)
        sel = _topk_mask(smask, k)

        sel16 = jnp.broadcast_to(sel[:, None], (n, 16))
        x, gsum, gmax = _tc_xnew(h_out, score16, sel16)
        gmean = gsum / float(k)
        layer_embs.append(jnp.concatenate([gmean, gmax], axis=1))

        if i + 1 < NUM_LAYERS:
            selp = jnp.pad(jnp.broadcast_to(sel[:, None], (n, 16)),
                           ((0, _NPAD - n), (0, 0)))
            ssrc = _gather16(selp, src)
            sdst = _gather16(selp, dst)
            evalid16 = ssrc * sdst * evalid[:, None]
            evalid = evalid16[:, 0]
        valid_n = sel
        n_cur = k

    out = (layer_embs[0] + layer_embs[1])[0]
    logits, probs = _mlp_head(out, params)
    return (logits, probs, label)


# speedup vs baseline: 1.2769x; 1.0151x over previous
import functools
import math

import jax
import jax.numpy as jnp
from jax import lax
from jax.experimental import pallas as pl
from jax.experimental.pallas import tpu as pltpu
from jax.experimental.pallas import tpu_sc as plsc

N_NODES = 10000
N_EDGES = 160000
HIDDEN = 128
HEADS = 2
NUM_LAYERS = 2
RATIO = 0.5
NEG_SLOPE = 0.2
NEG_BIG = -1e30

_NC, _NS = 2, 16
_NW = _NC * _NS
_NPAD = 10240
_RBLK = 1000
_EBLK = 2000

_sc_mesh = functools.partial(
    plsc.VectorSubcoreMesh, core_axis_name="c", subcore_axis_name="s")


def _split_chunks(total):
    per = total // _NW
    b = min(per, 128)
    return per, b, per // b, per % b


@functools.cache
def _gather_rows_kernel(E, D):
    per, B, nfull, tail = _split_chunks(E)

    scratch = [
        pltpu.VMEM((B,), jnp.int32),
        pltpu.VMEM((B, D), jnp.float32),
        pltpu.VMEM((max(tail, 8),), jnp.int32),
        pltpu.VMEM((max(tail, 8), D), jnp.float32),
        pltpu.SemaphoreType.DMA,
    ]

    @functools.partial(
        pl.kernel,
        out_type=jax.ShapeDtypeStruct((E, D), jnp.float32),
        mesh=_sc_mesh(),
        scratch_types=scratch,
    )
    def k(table_hbm, idx_hbm, out_hbm, idx_v, rows_v, idx_t, rows_t, sem):
        wid = lax.axis_index("s") * _NC + lax.axis_index("c")
        base = wid * per

        def step(c, carry):
            off = base + c * B
            pltpu.sync_copy(idx_hbm.at[pl.ds(off, B)], idx_v)
            pltpu.async_copy(table_hbm.at[idx_v], rows_v, sem).wait()
            pltpu.sync_copy(rows_v, out_hbm.at[pl.ds(off, B)])
            return carry

        lax.fori_loop(0, nfull, step, 0)
        if tail:
            off = base + nfull * B
            pltpu.sync_copy(idx_hbm.at[pl.ds(off, tail)], idx_t)
            pltpu.async_copy(table_hbm.at[idx_t], rows_t, sem).wait()
            pltpu.sync_copy(rows_t, out_hbm.at[pl.ds(off, tail)])

    return k


def _gather_rows(table, idx):
    E = idx.shape[0]
    return _gather_rows_kernel(E, table.shape[1])(table, idx)


@functools.cache
def _scatter_add_kernel(E, D):
    per, B, nfull, tail = _split_chunks(E)
    rpt = _NPAD // _NS
    zb = 128
    nz = rpt // zb

    scratch = [
        pltpu.VMEM((B,), jnp.int32),
        pltpu.VMEM((B, D), jnp.float32),
        pltpu.VMEM((max(tail, 8),), jnp.int32),
        pltpu.VMEM((max(tail, 8), D), jnp.float32),
        pltpu.VMEM((zb, D), jnp.float32),
        pltpu.VMEM_SHARED((_NPAD, D), jnp.float32),
        pltpu.SemaphoreType.DMA,
    ]
    assert tail in (0, max(tail, 8))

    @functools.partial(
        pl.kernel,
        out_type=jax.ShapeDtypeStruct((_NC, _NPAD, D), jnp.float32),
        mesh=_sc_mesh(),
        scratch_types=scratch,
    )
    def k(vals_hbm, idx_hbm, out_hbm, idx_v, rows_v, idx_t, rows_t,
          zero_v, acc_sh, sem):
        cid = lax.axis_index("c")
        sid = lax.axis_index("s")
        wid = sid * _NC + cid
        base = wid * per

        zv = jnp.zeros((16,), jnp.float32)

        def zrow(r, carry):
            def zcol(cc, carry2):
                zero_v[r, pl.ds(cc * 16, 16)] = zv
                return carry2
            return lax.fori_loop(0, D // 16, zcol, carry)

        lax.fori_loop(0, zb, zrow, 0)
        for z in range(nz):
            pltpu.sync_copy(zero_v,
                            acc_sh.at[pl.ds(sid * rpt + z * zb, zb)])
        plsc.subcore_barrier()

        def step(c, carry):
            off = base + c * B
            pltpu.sync_copy(idx_hbm.at[pl.ds(off, B)], idx_v)
            pltpu.sync_copy(vals_hbm.at[pl.ds(off, B)], rows_v)
            pltpu.sync_copy(rows_v, acc_sh.at[idx_v], add=True)
            return carry

        lax.fori_loop(0, nfull, step, 0)
        if tail:
            off = base + nfull * B
            pltpu.sync_copy(idx_hbm.at[pl.ds(off, tail)], idx_t)
            pltpu.sync_copy(vals_hbm.at[pl.ds(off, tail)], rows_t)
            pltpu.sync_copy(rows_t, acc_sh.at[idx_t], add=True)
        plsc.subcore_barrier()

        for z in range(nz):
            r0 = sid * rpt + z * zb
            pltpu.sync_copy(acc_sh.at[pl.ds(r0, zb)],
                            out_hbm.at[cid, pl.ds(r0, zb)])

    return k


def _scatter_add(vals, idx):
    parts = _scatter_add_kernel(idx.shape[0], vals.shape[1])(vals, idx)
    return parts


@functools.cache
def _gather16_kernel(E):
    per, B, nfull, tail = _split_chunks(E)
    rpt = _NPAD // _NS

    scratch = [
        pltpu.VMEM((B,), jnp.int32),
        pltpu.VMEM((B, 16), jnp.float32),
        pltpu.VMEM((max(tail, 8),), jnp.int32),
        pltpu.VMEM((max(tail, 8), 16), jnp.float32),
        pltpu.VMEM_SHARED((_NPAD, 16), jnp.float32),
        pltpu.SemaphoreType.DMA,
    ]

    @functools.partial(
        pl.kernel,
        out_type=jax.ShapeDtypeStruct((E, 16), jnp.float32),
        mesh=_sc_mesh(),
        scratch_types=scratch,
    )
    def k(tab_hbm, idx_hbm, out_hbm, idx_v, rows_v, idx_t, rows_t,
          tab_sh, sem):
        sid = lax.axis_index("s")
        wid = sid * _NC + lax.axis_index("c")
        base = wid * per
        pltpu.sync_copy(tab_hbm.at[pl.ds(sid * rpt, rpt)],
                        tab_sh.at[pl.ds(sid * rpt, rpt)])
        plsc.subcore_barrier()

        def step(c, carry):
            off = base + c * B
            pltpu.sync_copy(idx_hbm.at[pl.ds(off, B)], idx_v)
            pltpu.async_copy(tab_sh.at[idx_v], rows_v, sem).wait()
            pltpu.sync_copy(rows_v, out_hbm.at[pl.ds(off, B)])
            return carry

        lax.fori_loop(0, nfull, step, 0)
        if tail:
            off = base + nfull * B
            pltpu.sync_copy(idx_hbm.at[pl.ds(off, tail)], idx_t)
            pltpu.async_copy(tab_sh.at[idx_t], rows_t, sem).wait()
            pltpu.sync_copy(rows_t, out_hbm.at[pl.ds(off, tail)])

    return k


def _gather16(table16, idx):
    return _gather16_kernel(idx.shape[0])(table16, idx)


@functools.cache
def _gsg_plain_kernel(E, D):
    per, B, nfull, tail = _split_chunks(E)
    rpt = _NPAD // _NS
    zb = 128
    nz = rpt // zb

    scratch = [
        pltpu.VMEM((B,), jnp.int32),
        pltpu.VMEM((B,), jnp.int32),
        pltpu.VMEM((B, D), jnp.float32),
        pltpu.VMEM((max(tail, 8),), jnp.int32),
        pltpu.VMEM((max(tail, 8),), jnp.int32),
        pltpu.VMEM((max(tail, 8), D), jnp.float32),
        pltpu.VMEM((zb, D), jnp.float32),
        pltpu.VMEM_SHARED((_NPAD, D), jnp.float32),
        pltpu.SemaphoreType.DMA,
    ]

    @functools.partial(
        pl.kernel,
        out_type=jax.ShapeDtypeStruct((_NC, _NPAD, D), jnp.float32),
        mesh=_sc_mesh(),
        scratch_types=scratch,
    )
    def k(table_hbm, src_hbm, dst_hbm, out_hbm, sidx_v, didx_v, rows_v,
          sidx_t, didx_t, rows_t, zero_v, acc_sh, sem):
        cid = lax.axis_index("c")
        sid = lax.axis_index("s")
        wid = sid * _NC + cid
        base = wid * per

        zv = jnp.zeros((16,), jnp.float32)

        def zrow(r, carry):
            def zcol(cc, carry2):
                zero_v[r, pl.ds(cc * 16, 16)] = zv
                return carry2
            return lax.fori_loop(0, D // 16, zcol, carry)

        lax.fori_loop(0, zb, zrow, 0)
        for z in range(nz):
            pltpu.sync_copy(zero_v,
                            acc_sh.at[pl.ds(sid * rpt + z * zb, zb)])
        plsc.subcore_barrier()

        def step(c, carry):
            off = base + c * B
            pltpu.sync_copy(src_hbm.at[pl.ds(off, B)], sidx_v)
            pltpu.async_copy(table_hbm.at[sidx_v], rows_v, sem).wait()
            pltpu.sync_copy(dst_hbm.at[pl.ds(off, B)], didx_v)
            pltpu.sync_copy(rows_v, acc_sh.at[didx_v], add=True)
            return carry

        lax.fori_loop(0, nfull, step, 0)
        if tail:
            off = base + nfull * B
            pltpu.sync_copy(src_hbm.at[pl.ds(off, tail)], sidx_t)
            pltpu.async_copy(table_hbm.at[sidx_t], rows_t, sem).wait()
            pltpu.sync_copy(dst_hbm.at[pl.ds(off, tail)], didx_t)
            pltpu.sync_copy(rows_t, acc_sh.at[didx_t], add=True)
        plsc.subcore_barrier()

        for z in range(nz):
            r0 = sid * rpt + z * zb
            pltpu.sync_copy(acc_sh.at[pl.ds(r0, zb)],
                            out_hbm.at[cid, pl.ds(r0, zb)])

    return k


def _gather_scatter(table, src, dst):
    return _gsg_plain_kernel(src.shape[0], table.shape[1])(table, src, dst)


def _proj_body(h_ref, wl_ref, bl_ref, wr_ref, br_ref, la_ref, we_ref,
               att_ref, xl_ref, xr_ref, ews_ref):
    h = h_ref[...]
    xl = jnp.dot(h, wl_ref[...], preferred_element_type=jnp.float32) + bl_ref[...]
    xr = jnp.dot(h, wr_ref[...], preferred_element_type=jnp.float32) + br_ref[...]
    eel = jnp.dot(la_ref[...], we_ref[...], preferred_element_type=jnp.float32)
    m = xl + xr + eel
    m = jnp.where(m > 0, m, NEG_SLOPE * m) * att_ref[...]
    s0 = jnp.sum(m[:, :HIDDEN], axis=1, keepdims=True)
    s1 = jnp.sum(m[:, HIDDEN:], axis=1, keepdims=True)
    e = jnp.exp(jnp.concatenate([s0, s1], axis=1))
    ews_ref[...] = jnp.pad(e, ((0, 0), (0, 14)))
    xl_ref[...] = xl
    xr_ref[...] = xr


def _tc_proj(h, la, p):
    n = h.shape[0]
    grid = n // _RBLK
    din = h.shape[1]
    attv = p['att'].reshape(1, HEADS * HIDDEN)
    return pl.pallas_call(
        _proj_body,
        grid=(grid,),
        in_specs=[
            pl.BlockSpec((_RBLK, din), lambda i: (i, 0)),
            pl.BlockSpec((din, HEADS * HIDDEN), lambda i: (0, 0)),
            pl.BlockSpec((1, HEADS * HIDDEN), lambda i: (0, 0)),
            pl.BlockSpec((din, HEADS * HIDDEN), lambda i: (0, 0)),
            pl.BlockSpec((1, HEADS * HIDDEN), lambda i: (0, 0)),
            pl.BlockSpec((_RBLK, EMB16), lambda i: (i, 0)),
            pl.BlockSpec((EMB16, HEADS * HIDDEN), lambda i: (0, 0)),
            pl.BlockSpec((1, HEADS * HIDDEN), lambda i: (0, 0)),
        ],
        out_specs=[
            pl.BlockSpec((_RBLK, HEADS * HIDDEN), lambda i: (i, 0)),
            pl.BlockSpec((_RBLK, HEADS * HIDDEN), lambda i: (i, 0)),
            pl.BlockSpec((_RBLK, 16), lambda i: (i, 0)),
        ],
        out_shape=[
            jax.ShapeDtypeStruct((n, HEADS * HIDDEN), jnp.float32),
            jax.ShapeDtypeStruct((n, HEADS * HIDDEN), jnp.float32),
            jax.ShapeDtypeStruct((n, 16), jnp.float32),
        ],
    )(h, p['W_l'], p['b_l'].reshape(1, -1), p['W_r'], p['b_r'].reshape(1, -1),
      la, p['W_e'], attv)


EMB16 = 16


def _edge_body(xls_ref, xrd_ref, oh_ref, ev_ref, eemb_ref, we_ref, att_ref,
               ew_ref, v0_ref, v1_ref):
    eetab = jnp.dot(eemb_ref[...], we_ref[...],
                    preferred_element_type=jnp.float32)
    ee = jnp.dot(oh_ref[...], eetab, preferred_element_type=jnp.float32)
    xls = xls_ref[...]
    m = xls + xrd_ref[...] + ee
    m = jnp.where(m > 0, m, NEG_SLOPE * m) * att_ref[...]
    s0 = jnp.sum(m[:, :HIDDEN], axis=1, keepdims=True)
    s1 = jnp.sum(m[:, HIDDEN:], axis=1, keepdims=True)
    ev = ev_ref[...]
    e0 = jnp.exp(s0) * ev
    e1 = jnp.exp(s1) * ev
    ew_ref[...] = jnp.pad(jnp.concatenate([e0, e1], axis=1),
                          ((0, 0), (0, 14)))
    v0_ref[...] = xls[:, :HIDDEN] * e0
    v1_ref[...] = xls[:, HIDDEN:] * e1


def _tc_edge(xls, xrd, onehot, ev1, eemb, we, att):
    grid = N_EDGES // _EBLK
    attv = att.reshape(1, HEADS * HIDDEN)
    return pl.pallas_call(
        _edge_body,
        grid=(grid,),
        in_specs=[
            pl.BlockSpec((_EBLK, HEADS * HIDDEN), lambda i: (i, 0)),
            pl.BlockSpec((_EBLK, HEADS * HIDDEN), lambda i: (i, 0)),
            pl.BlockSpec((_EBLK, EMB16), lambda i: (i, 0)),
            pl.BlockSpec((_EBLK, 1), lambda i: (i, 0)),
            pl.BlockSpec((EMB16, EMB16), lambda i: (0, 0)),
            pl.BlockSpec((EMB16, HEADS * HIDDEN), lambda i: (0, 0)),
            pl.BlockSpec((1, HEADS * HIDDEN), lambda i: (0, 0)),
        ],
        out_specs=[
            pl.BlockSpec((_EBLK, 16), lambda i: (i, 0)),
            pl.BlockSpec((_EBLK, HIDDEN), lambda i: (i, 0)),
            pl.BlockSpec((_EBLK, HIDDEN), lambda i: (i, 0)),
        ],
        out_shape=[
            jax.ShapeDtypeStruct((N_EDGES, 16), jnp.float32),
            jax.ShapeDtypeStruct((N_EDGES, HIDDEN), jnp.float32),
            jax.ShapeDtypeStruct((N_EDGES, HIDDEN), jnp.float32),
        ],
    )(xls, xrd, onehot, ev1, eemb, we, attv)


def _combine_body(dp_ref, ews_ref, dr_ref):
    d0 = dp_ref[0, :, 0:1] + dp_ref[1, :, 0:1] + ews_ref[:, 0:1]
    d1 = dp_ref[0, :, 1:2] + dp_ref[1, :, 1:2] + ews_ref[:, 1:2]
    r0 = 1.0 / jnp.maximum(d0, 1e-16)
    r1 = 1.0 / jnp.maximum(d1, 1e-16)
    dr_ref[...] = jnp.pad(jnp.concatenate([r0, r1], axis=1),
                          ((0, 0), (0, 14)))


def _tc_combine(dparts, ews):
    n = ews.shape[0]
    grid = n // _RBLK
    return pl.pallas_call(
        _combine_body,
        grid=(grid,),
        in_specs=[
            pl.BlockSpec((2, _RBLK, 16), lambda i: (0, i, 0)),
            pl.BlockSpec((_RBLK, 16), lambda i: (i, 0)),
        ],
        out_specs=pl.BlockSpec((_RBLK, 16), lambda i: (i, 0)),
        out_shape=jax.ShapeDtypeStruct((n, 16), jnp.float32),
    )(dparts, ews)


def _post_body(p0_ref, p1_ref, xl_ref, ews_ref, dr_ref, b_ref, vm_ref,
               h_ref, hm_ref):
    xl = xl_ref[...]
    o0 = (p0_ref[0] + p0_ref[1] + xl[:, :HIDDEN] * ews_ref[:, 0:1]) \
        * dr_ref[:, 0:1]
    o1 = (p1_ref[0] + p1_ref[1] + xl[:, HIDDEN:] * ews_ref[:, 1:2]) \
        * dr_ref[:, 1:2]
    h = jnp.maximum(0.5 * (o0 + o1) + b_ref[...], 0.0)
    h_ref[...] = h
    hm_ref[...] = h * vm_ref[:, 0:1]


def _tc_post(p0, p1, xl, ews, denomr, bias, vmask16):
    n = xl.shape[0]
    grid = n // _RBLK
    return pl.pallas_call(
        _post_body,
        grid=(grid,),
        in_specs=[
            pl.BlockSpec((2, _RBLK, HIDDEN), lambda i: (0, i, 0)),
            pl.BlockSpec((2, _RBLK, HIDDEN), lambda i: (0, i, 0)),
            pl.BlockSpec((_RBLK, HEADS * HIDDEN), lambda i: (i, 0)),
            pl.BlockSpec((_RBLK, 16), lambda i: (i, 0)),
            pl.BlockSpec((_RBLK, 16), lambda i: (i, 0)),
            pl.BlockSpec((1, HIDDEN), lambda i: (0, 0)),
            pl.BlockSpec((_RBLK, 16), lambda i: (i, 0)),
        ],
        out_specs=[
            pl.BlockSpec((_RBLK, HIDDEN), lambda i: (i, 0)),
            pl.BlockSpec((_RBLK, HIDDEN), lambda i: (i, 0)),
        ],
        out_shape=[
            jax.ShapeDtypeStruct((n, HIDDEN), jnp.float32),
            jax.ShapeDtypeStruct((n, HIDDEN), jnp.float32),
        ],
    )(p0, p1, xl, ews, denomr, bias.reshape(1, HIDDEN), vmask16)


def _lookup_body(oh_ref, tab_ref, o_ref):
    o_ref[...] = jnp.dot(oh_ref[...], tab_ref[...],
                         preferred_element_type=jnp.float32)


def _tc_lookup(onehot, tab, blk):
    total, v = onehot.shape
    d = tab.shape[1]
    grid = total // blk
    return pl.pallas_call(
        _lookup_body,
        grid=(grid,),
        in_specs=[
            pl.BlockSpec((blk, v), lambda i: (i, 0)),
            pl.BlockSpec((v, d), lambda i: (0, 0)),
        ],
        out_specs=pl.BlockSpec((blk, d), lambda i: (i, 0)),
        out_shape=jax.ShapeDtypeStruct((total, d), jnp.float32),
    )(onehot, tab)


def _lvals_body(oh_ref, ev_ref, eemb_ref, o_ref):
    ee = jnp.dot(oh_ref[...], eemb_ref[...],
                 preferred_element_type=jnp.float32)
    ev = ev_ref[...]
    o_ref[...] = jnp.pad(jnp.concatenate([ev, ee * ev], axis=1),
                         ((0, 0), (0, 15)))


def _tc_lvals(onehot, ev1, eemb):
    grid = N_EDGES // _EBLK
    return pl.pallas_call(
        _lvals_body,
        grid=(grid,),
        in_specs=[
            pl.BlockSpec((_EBLK, EMB16), lambda i: (i, 0)),
            pl.BlockSpec((_EBLK, 1), lambda i: (i, 0)),
            pl.BlockSpec((EMB16, EMB16), lambda i: (0, 0)),
        ],
        out_specs=pl.BlockSpec((_EBLK, 32), lambda i: (i, 0)),
        out_shape=jax.ShapeDtypeStruct((N_EDGES, 32), jnp.float32),
    )(onehot, ev1, eemb)


def _score_body(ap_ref, h_ref, wrel_ref, wroot_ref, brel_ref, s_ref):
    a = ap_ref[0] + ap_ref[1]
    s = (jnp.sum(a * wrel_ref[...], axis=1, keepdims=True)
         + jnp.sum(h_ref[...] * wroot_ref[...], axis=1, keepdims=True)
         + brel_ref[0:1, 0:1])
    s_ref[...] = jnp.pad(jnp.tanh(s), ((0, 0), (0, 15)))


def _tc_score(aparts, h_out, wrel, wroot, brel):
    n = h_out.shape[0]
    grid = n // _RBLK
    return pl.pallas_call(
        _score_body,
        grid=(grid,),
        in_specs=[
            pl.BlockSpec((2, _RBLK, HIDDEN), lambda i: (0, i, 0)),
            pl.BlockSpec((_RBLK, HIDDEN), lambda i: (i, 0)),
            pl.BlockSpec((1, HIDDEN), lambda i: (0, 0)),
            pl.BlockSpec((1, HIDDEN), lambda i: (0, 0)),
            pl.BlockSpec((1, HIDDEN), lambda i: (0, 0)),
        ],
        out_specs=pl.BlockSpec((_RBLK, 16), lambda i: (i, 0)),
        out_shape=jax.ShapeDtypeStruct((n, 16), jnp.float32),
    )(aparts, h_out, wrel.reshape(1, HIDDEN), wroot.reshape(1, HIDDEN),
      jnp.broadcast_to(brel.reshape(1, 1), (1, HIDDEN)))


@functools.cache
def _topk_kernel(k):
    rows = _NPAD // 128

    def body(s_ref, sel_ref):
        f = s_ref[...]
        u = lax.bitcast_convert_type(f, jnp.uint32)
        sign = u >= jnp.uint32(0x80000000)
        ukey = u ^ jnp.where(sign, jnp.uint32(0xFFFFFFFF),
                             jnp.uint32(0x80000000))

        def count_ge(t):
            return jnp.sum((ukey >= t).astype(jnp.int32))

        def bs1(_, carry):
            lo, hi = carry
            mid = lo + (hi - lo) // jnp.uint32(2)
            c = count_ge(mid)
            big = c >= k
            return (jnp.where(big, mid, lo), jnp.where(big, hi, mid))

        lo, hi = lax.fori_loop(
            0, 33, bs1, (jnp.uint32(0), jnp.uint32(0xFFFFFFFF)))
        v = lo
        c1 = jnp.sum((ukey > v).astype(jnp.int32))
        r = k - c1
        eq = ukey == v
        idx = (lax.broadcasted_iota(jnp.int32, (rows, 128), 0) * 128
               + lax.broadcasted_iota(jnp.int32, (rows, 128), 1))

        def bs2(_, carry):
            lo2, hi2 = carry
            mid = lo2 + (hi2 - lo2) // 2
            c = jnp.sum((eq & (idx <= mid)).astype(jnp.int32))
            ok = c >= r
            return (jnp.where(ok, lo2, mid), jnp.where(ok, mid, hi2))

        lo2, hi2 = lax.fori_loop(0, 15, bs2, (jnp.int32(-1),
                                              jnp.int32(_NPAD - 1)))
        j = hi2
        sel = (ukey > v) | (eq & (idx <= j))
        sel_ref[...] = sel.astype(jnp.float32)

    return pl.pallas_call(
        body,
        out_shape=jax.ShapeDtypeStruct((rows, 128), jnp.float32),
    )


def _topk_mask(smask, k):
    rows = _NPAD // 128
    pad = jnp.full((_NPAD - N_NODES,), -2.0, jnp.float32)
    s80 = jnp.concatenate([smask, pad]).reshape(rows, 128)
    sel = _topk_kernel(k)(s80)
    return sel.reshape(-1)[:N_NODES]


def _xnew_body(h_ref, s_ref, sel_ref, xn_ref, gs_ref, gm_ref):
    pid = pl.program_id(0)
    sc = s_ref[:, 0:1]
    se = sel_ref[:, 0:1]
    xn = h_ref[...] * sc * se
    xn_ref[...] = xn

    @pl.when(pid == 0)
    def _():
        gs_ref[...] = jnp.zeros_like(gs_ref)
        gm_ref[...] = jnp.full_like(gm_ref, NEG_BIG)

    gs_ref[...] += jnp.sum(xn, axis=0, keepdims=True)
    masked = jnp.where(se > 0, xn, NEG_BIG)
    gm_ref[...] = jnp.maximum(gm_ref[...], jnp.max(masked, axis=0,
                                                   keepdims=True))


def _tc_xnew(h_out, score16, sel16):
    n = h_out.shape[0]
    grid = n // _RBLK
    return pl.pallas_call(
        _xnew_body,
        grid=(grid,),
        in_specs=[
            pl.BlockSpec((_RBLK, HIDDEN), lambda i: (i, 0)),
            pl.BlockSpec((_RBLK, 16), lambda i: (i, 0)),
            pl.BlockSpec((_RBLK, 16), lambda i: (i, 0)),
        ],
        out_specs=[
            pl.BlockSpec((_RBLK, HIDDEN), lambda i: (i, 0)),
            pl.BlockSpec((1, HIDDEN), lambda i: (0, 0)),
            pl.BlockSpec((1, HIDDEN), lambda i: (0, 0)),
        ],
        out_shape=[
            jax.ShapeDtypeStruct((n, HIDDEN), jnp.float32),
            jax.ShapeDtypeStruct((1, HIDDEN), jnp.float32),
            jax.ShapeDtypeStruct((1, HIDDEN), jnp.float32),
        ],
    )(h_out, score16, sel16)


def _head_body(x_ref, w1_ref, b1_ref, w2_ref, b2_ref, w3_ref, b3_ref,
               logits_ref, probs_ref):
    x = x_ref[...]
    h1 = jnp.maximum(jnp.dot(x, w1_ref[...],
                             preferred_element_type=jnp.float32)
                     + b1_ref[...], 0.0)
    h2 = jnp.maximum(jnp.dot(h1, w2_ref[...],
                             preferred_element_type=jnp.float32)
                     + b2_ref[...], 0.0)
    logits = (jnp.dot(h2, w3_ref[...], preferred_element_type=jnp.float32)
              + b3_ref[...])
    ncls = lax.broadcasted_iota(jnp.int32, logits.shape, 1) < 2
    lm = jnp.where(ncls, logits, NEG_BIG)
    mx = jnp.max(lm, axis=1, keepdims=True)
    ew = jnp.where(ncls, jnp.exp(lm - mx), 0.0)
    probs_ref[...] = ew / jnp.sum(ew, axis=1, keepdims=True)
    logits_ref[...] = logits


def _mlp_head(out_vec, params):
    x = jnp.zeros((8, 2 * HIDDEN), jnp.float32).at[0].set(out_vec)
    logits, probs = pl.pallas_call(
        _head_body,
        out_shape=(jax.ShapeDtypeStruct((8, 8), jnp.float32),
                   jax.ShapeDtypeStruct((8, 8), jnp.float32)),
    )(x, params['lin1_W'], params['lin1_b'].reshape(1, -1),
      params['lin2_W'], params['lin2_b'].reshape(1, -1),
      jnp.pad(params['lin3_W'], ((0, 0), (0, 6))),
      jnp.pad(params['lin3_b'], (0, 6)).reshape(1, -1))
    return logits[0:1, 0:2], probs[0:1, 0:2]


def kernel(x, edge_index, edge_attr, node_attr, random_walk_pe, batch,
           label, params):
    n = x.shape[0]
    src, dst = edge_index[0], edge_index[1]

    onehot = (edge_attr[:, None] == jnp.arange(EMB16, dtype=edge_attr.dtype)
              ).astype(jnp.float32)
    onehot_n = (node_attr[:, None]
                == jnp.arange(32, dtype=node_attr.dtype)).astype(jnp.float32)
    na_emb = _tc_lookup(onehot_n, params['node_emb'], _RBLK)

    evalid = jnp.ones((N_EDGES,), jnp.float32)
    valid_n = jnp.ones((n,), jnp.float32)
    rwpe = random_walk_pe
    n_cur = n
    layer_embs = []
    for i in range(NUM_LAYERS):
        cp = params['convs'][i]
        pp = params['pools'][i]

        h = jnp.concatenate([x, rwpe, na_emb], axis=1)

        vals32 = _tc_lvals(onehot, evalid[:, None], params['edge_emb'])
        dl = _scatter_add(vals32, dst)
        degloop = (dl[0] + dl[1])[:n]
        deg = degloop[:, 0:1]
        loop_attr = degloop[:, 1:17] / jnp.maximum(deg, 1.0)

        xl, xr, ews = _tc_proj(h, loop_attr, cp)

        xls = _gather_rows(xl, src)
        xrd = _gather_rows(xr, dst)

        ew, v0, v1 = _tc_edge(xls, xrd, onehot, evalid[:, None],
                              params['edge_emb'], cp['W_e'], cp['att'])

        dparts = _scatter_add(ew, dst)
        denomr = _tc_combine(dparts[:, :n], ews)

        p0 = _scatter_add(v0, dst)[:, :n]
        p1 = _scatter_add(v1, dst)[:, :n]
        vmask16 = jnp.broadcast_to(valid_n[:, None], (n, 16))
        h_out, hm = _tc_post(p0, p1, xl, ews, denomr, cp['bias'], vmask16)

        aparts = _gather_scatter(hm, src, dst)[:, :n]
        score16 = _tc_score(aparts, h_out, pp['W_rel'], pp['W_root'],
                            pp['b_rel'])

        k = int(math.ceil(RATIO * n_cur))
        smask = jnp.where(valid_n > 0, score16[:, 0], -2.0)
        sel = _topk_mask(smask, k)

        sel16 = jnp.broadcast_to(sel[:, None], (n, 16))
        x, gsum, gmax = _tc_xnew(h_out, score16, sel16)
        gmean = gsum / float(k)
        layer_embs.append(jnp.concatenate([gmean, gmax], axis=1))

        if i + 1 < NUM_LAYERS:
            selp = jnp.pad(jnp.broadcast_to(sel[:, None], (n, 16)),
                           ((0, _NPAD - n), (0, 0)))
            ssrc = _gather16(selp, src)
            sdst = _gather16(selp, dst)
            evalid16 = ssrc * sdst * evalid[:, None]
            evalid = evalid16[:, 0]
        valid_n = sel
        n_cur = k

    out = (layer_embs[0] + layer_embs[1])[0]
    logits, probs = _mlp_head(out, params)
    return (logits, probs, label)
